# Initial kernel scaffold; baseline (speedup 1.0000x reference)
import jax
import jax.numpy as jnp
from jax.experimental import pallas as pl


def kernel(data_imp, val, mis, index):
    raise NotImplementedError("write your pallas kernel here")


# speedup vs baseline: 2.5384x; 2.5384x over previous
import functools

import jax
import jax.numpy as jnp
from jax import lax
from jax.experimental import pallas as pl
from jax.experimental.pallas import tpu as pltpu
from jax.experimental.pallas import tpu_sc as plsc

_L = 16
_CH = 16
_NBUF = 2
_IC = 2048


@functools.lru_cache(maxsize=None)
def _make_sc_kernel(n_rows, row_len, b, d):
    nw = 32
    per_w = b // nw
    nch = per_w // _CH
    own_vr = per_w // _L
    kd = row_len // d
    d_pad = max(d, 128)

    mesh = plsc.VectorSubcoreMesh(core_axis_name="c", subcore_axis_name="s")

    @functools.partial(
        pl.kernel,
        out_type=(),
        mesh=mesh,
        compiler_params=pltpu.CompilerParams(needs_layout_passes=False),
        scratch_types=[
            pltpu.VMEM((_IC,), jnp.int32),
            pltpu.VMEM((n_rows,), jnp.int32),
            pltpu.VMEM((nch, _CH), jnp.int32),
            pltpu.VMEM((nch, _CH), jnp.int32),
            pltpu.VMEM((_NBUF, _CH, row_len), jnp.float32),
            pltpu.VMEM((_NBUF, _CH, row_len), jnp.float32),
            pltpu.VMEM((_NBUF, _CH, d_pad), jnp.float32),
            pltpu.VMEM((_NBUF, _CH, row_len), jnp.float32),
            pltpu.SemaphoreType.DMA,
            pltpu.SemaphoreType.DMA,
            pltpu.SemaphoreType.DMA((_NBUF,)),
            pltpu.SemaphoreType.DMA((_NBUF,)),
            pltpu.SemaphoreType.DMA((_NBUF,)),
            pltpu.SemaphoreType.DMA((_NBUF,)),
        ],
    )
    def sc_kernel(buf, valr, misr, idxr, idx2r,
                  idxc, m, scidx, wb, gb, vb, mb, ob,
                  sem_i, sem_x, sem_g, sem_v, sem_m, sem_s):
        cid = lax.axis_index("c")
        sid = lax.axis_index("s")
        wid = sid * 2 + cid
        base_vr = wid * own_vr

        iota = lax.iota(jnp.int32, _L)
        onehot = jnp.int32(1) << iota

        pltpu.make_async_copy(idx2r.at[pl.ds(base_vr, own_vr)], scidx, sem_x).start()

        def win_inner(i, b0):
            idxv = idxc[pl.ds(i * _L, _L)]
            plsc.store_scatter(m, [idxv], jnp.zeros((_L,), jnp.int32))
            plsc.addupdate_scatter(m, [idxv], onehot)
            lanes = plsc.load_gather(m, [idxv])
            msb = (plsc.bitcast(lanes.astype(jnp.float32), jnp.int32) >> 23) - 127
            plsc.store_scatter(m, [idxv], b0 + i * _L + iota, mask=(iota == msb))
            return b0

        def win_outer(ci, carry):
            cp = pltpu.make_async_copy(
                idxr.at[pl.ds(ci * _IC, _IC)], idxc, sem_i)
            cp.start()
            cp.wait()
            lax.fori_loop(0, _IC // _L, win_inner, ci * _IC)
            return carry

        lax.fori_loop(0, b // _IC, win_outer, None)

        pltpu.make_async_copy(idx2r.at[pl.ds(base_vr, own_vr)], scidx, sem_x).wait()
        for r in range(own_vr):
            idxv = scidx[r, pl.ds(0, _L)]
            wb[r, :] = plsc.load_gather(m, [idxv])

        def g_cp(c, s):
            return pltpu.make_async_copy(buf.at[scidx.at[c]], gb.at[s], sem_g.at[s])

        def v_cp(c, s):
            return pltpu.make_async_copy(valr.at[wb.at[c]], vb.at[s], sem_v.at[s])

        def m_cp(c, s):
            return pltpu.make_async_copy(misr.at[wb.at[c]], mb.at[s], sem_m.at[s])

        def s_cp(c, s):
            return pltpu.make_async_copy(ob.at[s], buf.at[scidx.at[c]], sem_s.at[s])

        def start_inputs(c, s):
            g_cp(c, s).start()
            v_cp(c, s).start()
            m_cp(c, s).start()

        for s in range(_NBUF):
            start_inputs(s, s)

        def chunk_body(it, carry):
            for s in range(_NBUF):
                c = it * _NBUF + s
                g_cp(c, s).wait()
                v_cp(c, s).wait()
                m_cp(c, s).wait()
                for j in range(_CH):
                    for q in range(d // _L):
                        msk = mb[s, j, pl.ds(q * _L, _L)] != 0.0
                        for kk in range(kd):
                            off = kk * d + q * _L
                            ob[s, j, pl.ds(off, _L)] = jnp.where(
                                msk,
                                gb[s, j, pl.ds(off, _L)],
                                vb[s, j, pl.ds(off, _L)],
                            )

                @pl.when(c >= _NBUF)
                def _():
                    s_cp(c - _NBUF, s).wait()

                s_cp(c, s).start()

                @pl.when(c + _NBUF < nch)
                def _():
                    start_inputs(c + _NBUF, s)
            return carry

        lax.fori_loop(0, nch // _NBUF, chunk_body, None)

        for s in range(_NBUF):
            s_cp(nch - _NBUF + s, s).wait()

    return sc_kernel


def kernel(data_imp, val, mis, index):
    b, k, d = val.shape
    n_rows = data_imp.shape[0] // k
    row_len = k * d

    d2 = data_imp.reshape(n_rows, row_len)
    v2 = val.reshape(b, row_len)
    m2 = mis.astype(jnp.float32)
    if d < 128:
        m2 = jnp.pad(m2, ((0, 0), (0, 128 - d)))
    idx = index.astype(jnp.int32)
    idx2 = idx.reshape(b // _L, _L)

    sck = _make_sc_kernel(n_rows, row_len, b, d)
    buf = jax.new_ref(d2)
    sck(buf, v2, m2, idx, idx2)
    return buf[...].reshape(data_imp.shape)
